# baseline (speedup 1.0000x reference)
import functools

import jax
import jax.numpy as jnp
from jax import lax
from jax.experimental import pallas as pl
from jax.experimental.pallas import tpu as pltpu
from jax.experimental.pallas import tpu_sc as plsc

ROWS = 128
COLS = 32768
NC = 2
NS = 16
L = 16
NW = NC * NS
CPW = COLS // NW
CHUNK = 256
NCHUNK = CPW // CHUNK
G = CHUNK // L


def _sc_body(x_hbm, out_hbm, buf0, buf1, idx_v, sem0, sem1):
    wid = lax.axis_index("s") * NC + lax.axis_index("c")
    base = wid * CPW
    bufs = (buf0, buf1)
    sems = (sem0, sem1)

    def src(ci):
        return x_hbm.at[:, pl.ds(base + ci * CHUNK, CHUNK)]

    copies = [None] * NCHUNK
    copies[0] = pltpu.async_copy(src(0), bufs[0], sems[0])
    for ci in range(NCHUNK):
        if ci + 1 < NCHUNK:
            copies[ci + 1] = pltpu.async_copy(
                src(ci + 1), bufs[(ci + 1) % 2], sems[(ci + 1) % 2])
        copies[ci].wait()
        buf = bufs[ci % 2]

        maxv0 = tuple(buf[0, pl.ds(g * L, L)] for g in range(G))
        maxi0 = tuple(jnp.zeros((L,), jnp.int32) for _ in range(G))

        def row_step(r, carry, buf=buf):
            mv, mi = carry
            ridx = jnp.full((L,), r, jnp.int32)
            nmv, nmi = [], []
            for g in range(G):
                v = buf[r, pl.ds(g * L, L)]
                gt = v > mv[g]
                nmv.append(jnp.where(gt, v, mv[g]))
                nmi.append(jnp.where(gt, ridx, mi[g]))
            return tuple(nmv), tuple(nmi)

        _, maxi = lax.fori_loop(1, ROWS, row_step, (maxv0, maxi0))
        for g in range(G):
            idx_v[pl.ds(ci * CHUNK + g * L, L)] = maxi[g]

    pltpu.sync_copy(idx_v, out_hbm.at[pl.ds(base, CPW)])


@jax.jit
def _argmax_sc(x):
    mesh = plsc.VectorSubcoreMesh(core_axis_name="c", subcore_axis_name="s")
    f = pl.kernel(
        _sc_body,
        out_type=jax.ShapeDtypeStruct((COLS,), jnp.int32),
        mesh=mesh,
        scratch_types=[
            pltpu.VMEM((ROWS, CHUNK), jnp.float32),
            pltpu.VMEM((ROWS, CHUNK), jnp.float32),
            pltpu.VMEM((CPW,), jnp.int32),
            pltpu.SemaphoreType.DMA,
            pltpu.SemaphoreType.DMA,
        ],
    )
    return f(x)


def kernel(x):
    return _argmax_sc(x).reshape(1, COLS).astype(jnp.int64)
